# baseline (speedup 1.0000x reference)
import functools

import jax
import jax.numpy as jnp
from jax import lax
from jax.experimental import pallas as pl
from jax.experimental.pallas import tpu as pltpu
from jax.experimental.pallas import tpu_sc as plsc

D = 128
NS = 16
CHUNK = 128


def _sc_gather_1core(ids2, tab):
    n_chunks = ids2.shape[0]
    B = n_chunks * CHUNK
    kpw = n_chunks // NS
    mesh = plsc.VectorSubcoreMesh(
        core_axis_name="c", subcore_axis_name="s",
        num_cores=1, num_subcores=NS)

    @functools.partial(
        pl.kernel,
        out_type=jax.ShapeDtypeStruct((B, D), jnp.float32),
        mesh=mesh,
        scratch_types=(
            [pltpu.VMEM((kpw, CHUNK), jnp.int32)]
            + [pltpu.VMEM((CHUNK, D), jnp.float32) for _ in range(kpw)]
            + [pltpu.SemaphoreType.DMA for _ in range(2 * kpw)]
        ),
    )
    def k(ids_hbm, tab_hbm, out_hbm, idx_v, *rest):
        bufs = rest[:kpw]
        gsem = rest[kpw:2 * kpw]
        wsem = rest[2 * kpw:]
        wid = lax.axis_index("s")
        cbase = wid * kpw
        pltpu.sync_copy(ids_hbm.at[pl.ds(cbase, kpw)], idx_v)
        gathers = [
            pltpu.async_copy(tab_hbm.at[idx_v.at[j]], bufs[j], gsem[j])
            for j in range(kpw)
        ]
        writes = []
        for j in range(kpw):
            gathers[j].wait()
            writes.append(pltpu.async_copy(
                bufs[j], out_hbm.at[pl.ds((cbase + j) * CHUNK, CHUNK)],
                wsem[j]))
        for w in writes:
            w.wait()

    return k(ids2, tab)


def _tc_project2(x0, x1, W, b):
    B2 = x0.shape[0]
    BM = 2048
    dn = (((1,), (1,)), ((), ()))

    def body(x0_ref, x1_ref, w_ref, b_ref, o_ref):
        o_ref[0] = lax.dot_general(
            x0_ref[...], w_ref[...], dn,
            preferred_element_type=jnp.float32) + b_ref[...]
        o_ref[1] = lax.dot_general(
            x1_ref[...], w_ref[...], dn,
            preferred_element_type=jnp.float32) + b_ref[...]

    return pl.pallas_call(
        body,
        grid=(B2 // BM,),
        in_specs=[
            pl.BlockSpec((BM, D), lambda i: (i, 0)),
            pl.BlockSpec((BM, D), lambda i: (i, 0)),
            pl.BlockSpec((D, D), lambda i: (0, 0)),
            pl.BlockSpec((1, D), lambda i: (0, 0)),
        ],
        out_specs=pl.BlockSpec((2, BM, D), lambda i: (0, i, 0)),
        out_shape=jax.ShapeDtypeStruct((2, B2, D), jnp.float32),
    )(x0, x1, W, b.reshape(1, D))


def kernel(user_ids, item_ids, user_embedding, item_embedding,
           W_user, b_user, W_item, b_item):
    B = user_ids.shape[0]
    B2 = B // 2
    nch = B2 // CHUNK
    uids = user_ids.astype(jnp.int32)
    iids = item_ids.astype(jnp.int32)
    u0 = uids[:B2].reshape(nch, CHUNK)
    u1 = uids[B2:].reshape(nch, CHUNK)
    i0 = iids[:B2].reshape(nch, CHUNK)
    i1 = iids[B2:].reshape(nch, CHUNK)
    gu0 = _sc_gather_1core(u0, user_embedding)
    gu1 = _sc_gather_1core(u1, user_embedding)
    gi0 = _sc_gather_1core(i0, item_embedding)
    gi1 = _sc_gather_1core(i1, item_embedding)
    ou = _tc_project2(gu0, gu1, W_user, b_user).reshape(B, D)
    oi = _tc_project2(gi0, gi1, W_item, b_item).reshape(B, D)
    return (ou, oi)


# speedup vs baseline: 1.2575x; 1.2575x over previous
import functools

import jax
import jax.numpy as jnp
from jax import lax
from jax.experimental import pallas as pl
from jax.experimental.pallas import tpu as pltpu
from jax.experimental.pallas import tpu_sc as plsc

D = 128
NC, NS = 2, 16
NW = NC * NS
CHUNK = 128


def _sc_gather_half(uids2, iids2, utab, itab):
    n_chunks = uids2.shape[0]
    B2 = n_chunks * CHUNK
    kpw = n_chunks // NW
    mesh = plsc.VectorSubcoreMesh(
        core_axis_name="c", subcore_axis_name="s",
        num_cores=NC, num_subcores=NS)

    @functools.partial(
        pl.kernel,
        out_type=jax.ShapeDtypeStruct((2, B2, D), jnp.float32),
        mesh=mesh,
        scratch_types=(
            [pltpu.VMEM((kpw, CHUNK), jnp.int32) for _ in range(2)]
            + [pltpu.VMEM((CHUNK, D), jnp.float32) for _ in range(2 * kpw)]
            + [pltpu.SemaphoreType.DMA for _ in range(4 * kpw)]
        ),
    )
    def k(uid_hbm, iid_hbm, utab_hbm, itab_hbm, out_hbm, *rest):
        idx = rest[:2]
        bufs = rest[2:2 + 2 * kpw]
        gsem = rest[2 + 2 * kpw:2 + 4 * kpw]
        wsem = rest[2 + 4 * kpw:]
        wid = lax.axis_index("s") * NC + lax.axis_index("c")
        cbase = wid * kpw
        pltpu.sync_copy(uid_hbm.at[pl.ds(cbase, kpw)], idx[0])
        pltpu.sync_copy(iid_hbm.at[pl.ds(cbase, kpw)], idx[1])
        gathers = []
        for p, tab in enumerate((utab_hbm, itab_hbm)):
            for j in range(kpw):
                n = p * kpw + j
                gathers.append(pltpu.async_copy(
                    tab.at[idx[p].at[j]], bufs[n], gsem[n]))
        writes = []
        for p in range(2):
            for j in range(kpw):
                n = p * kpw + j
                gathers[n].wait()
                writes.append(pltpu.async_copy(
                    bufs[n],
                    out_hbm.at[p, pl.ds((cbase + j) * CHUNK, CHUNK)],
                    wsem[n]))
        for w in writes:
            w.wait()

    return k(uids2, iids2, utab, itab)


def _tc_project_half(X, Wc, bc, half, nhalves, B, prev=None):
    B2 = X.shape[1]
    BM = 2048
    nblk = B2 // BM
    blk0 = half * nblk
    dn = (((1,), (1,)), ((), ()))

    def body(x_ref, w_ref, b_ref, *rest):
        ou_ref, oi_ref = rest[-2], rest[-1]
        x = jnp.concatenate([x_ref[0], x_ref[1]], axis=1)
        r = lax.dot_general(x, w_ref[...], dn,
                            preferred_element_type=jnp.float32) + b_ref[...]
        ou_ref[...] = r[:, :D]
        oi_ref[...] = r[:, D:]

    in_specs = [
        pl.BlockSpec((2, BM, D), lambda i: (0, i, 0)),
        pl.BlockSpec((2 * D, 2 * D), lambda i: (0, 0)),
        pl.BlockSpec((1, 2 * D), lambda i: (0, 0)),
    ]
    inputs = [X, Wc, bc]
    aliases = {}
    if prev is not None:
        in_specs += [
            pl.BlockSpec((BM, D), lambda i, b0=blk0: (b0 + i, 0)),
            pl.BlockSpec((BM, D), lambda i, b0=blk0: (b0 + i, 0)),
        ]
        inputs += [prev[0], prev[1]]
        aliases = {3: 0, 4: 1}

    return pl.pallas_call(
        body,
        grid=(nblk,),
        in_specs=in_specs,
        out_specs=[
            pl.BlockSpec((BM, D), lambda i, b0=blk0: (b0 + i, 0)),
            pl.BlockSpec((BM, D), lambda i, b0=blk0: (b0 + i, 0)),
        ],
        out_shape=[
            jax.ShapeDtypeStruct((B, D), jnp.float32),
            jax.ShapeDtypeStruct((B, D), jnp.float32),
        ],
        input_output_aliases=aliases,
    )(*inputs)


def kernel(user_ids, item_ids, user_embedding, item_embedding,
           W_user, b_user, W_item, b_item):
    B = user_ids.shape[0]
    B2 = B // 2
    nch = B2 // CHUNK
    uids = user_ids.astype(jnp.int32)
    iids = item_ids.astype(jnp.int32)

    Wc = jnp.zeros((2 * D, 2 * D), jnp.float32)
    Wc = Wc.at[:D, :D].set(W_user).at[D:, D:].set(W_item)
    bc = jnp.concatenate([b_user, b_item]).reshape(1, 2 * D)

    X0 = _sc_gather_half(uids[:B2].reshape(nch, CHUNK),
                         iids[:B2].reshape(nch, CHUNK),
                         user_embedding, item_embedding)
    X1 = _sc_gather_half(uids[B2:].reshape(nch, CHUNK),
                         iids[B2:].reshape(nch, CHUNK),
                         user_embedding, item_embedding)
    out0 = _tc_project_half(X0, Wc, bc, 0, 2, B)
    ou, oi = _tc_project_half(X1, Wc, bc, 1, 2, B, prev=out0)
    return (ou, oi)


# speedup vs baseline: 1.3041x; 1.0371x over previous
import functools

import jax
import jax.numpy as jnp
from jax import lax
from jax.experimental import pallas as pl
from jax.experimental.pallas import tpu as pltpu
from jax.experimental.pallas import tpu_sc as plsc

D = 128
NC, NS = 2, 16
NW = NC * NS
CHUNK = 128
SPLIT = 12288


def _sc_gather_piece(uids3, iids3, utab, itab):
    kpw = uids3.shape[1]
    rows = NW * kpw * CHUNK
    mesh = plsc.VectorSubcoreMesh(
        core_axis_name="c", subcore_axis_name="s",
        num_cores=NC, num_subcores=NS)

    @functools.partial(
        pl.kernel,
        out_type=jax.ShapeDtypeStruct((2, rows, D), jnp.float32),
        mesh=mesh,
        scratch_types=(
            [pltpu.VMEM((kpw, CHUNK), jnp.int32) for _ in range(2)]
            + [pltpu.VMEM((CHUNK, D), jnp.float32) for _ in range(2 * kpw)]
            + [pltpu.SemaphoreType.DMA for _ in range(4 * kpw)]
        ),
    )
    def k(uid_hbm, iid_hbm, utab_hbm, itab_hbm, out_hbm, *rest):
        idx = rest[:2]
        bufs = rest[2:2 + 2 * kpw]
        gsem = rest[2 + 2 * kpw:2 + 4 * kpw]
        wsem = rest[2 + 4 * kpw:]
        wid = lax.axis_index("s") * NC + lax.axis_index("c")
        cbase = wid * kpw
        pltpu.sync_copy(uid_hbm.at[wid], idx[0])
        pltpu.sync_copy(iid_hbm.at[wid], idx[1])
        gathers = []
        for p, tab in enumerate((utab_hbm, itab_hbm)):
            for j in range(kpw):
                n = p * kpw + j
                gathers.append(pltpu.async_copy(
                    tab.at[idx[p].at[j]], bufs[n], gsem[n]))
        writes = []
        for p in range(2):
            for j in range(kpw):
                n = p * kpw + j
                gathers[n].wait()
                writes.append(pltpu.async_copy(
                    bufs[n],
                    out_hbm.at[p, pl.ds((cbase + j) * CHUNK, CHUNK)],
                    wsem[n]))
        for w in writes:
            w.wait()

    return k(uids3, iids3, utab, itab)


def _tc_project_piece(X, Wu, bu, Wi, bi, blk0, B, prev=None):
    rows = X.shape[1]
    BM = 2048
    nblk = rows // BM
    dn = (((1,), (1,)), ((), ()))

    def body(x_ref, wu_ref, bu_ref, wi_ref, bi_ref, *rest):
        ou_ref, oi_ref = rest[-2], rest[-1]
        ou_ref[...] = lax.dot_general(
            x_ref[0], wu_ref[...], dn,
            preferred_element_type=jnp.float32) + bu_ref[...]
        oi_ref[...] = lax.dot_general(
            x_ref[1], wi_ref[...], dn,
            preferred_element_type=jnp.float32) + bi_ref[...]

    in_specs = [
        pl.BlockSpec((2, BM, D), lambda i: (0, i, 0)),
        pl.BlockSpec((D, D), lambda i: (0, 0)),
        pl.BlockSpec((1, D), lambda i: (0, 0)),
        pl.BlockSpec((D, D), lambda i: (0, 0)),
        pl.BlockSpec((1, D), lambda i: (0, 0)),
    ]
    inputs = [X, Wu, bu.reshape(1, D), Wi, bi.reshape(1, D)]
    aliases = {}
    if prev is not None:
        in_specs += [
            pl.BlockSpec((BM, D), lambda i, b0=blk0: (b0 + i, 0)),
            pl.BlockSpec((BM, D), lambda i, b0=blk0: (b0 + i, 0)),
        ]
        inputs += [prev[0], prev[1]]
        aliases = {5: 0, 6: 1}

    return pl.pallas_call(
        body,
        grid=(nblk,),
        in_specs=in_specs,
        out_specs=[
            pl.BlockSpec((BM, D), lambda i, b0=blk0: (b0 + i, 0)),
            pl.BlockSpec((BM, D), lambda i, b0=blk0: (b0 + i, 0)),
        ],
        out_shape=[
            jax.ShapeDtypeStruct((B, D), jnp.float32),
            jax.ShapeDtypeStruct((B, D), jnp.float32),
        ],
        input_output_aliases=aliases,
    )(*inputs)


def kernel(user_ids, item_ids, user_embedding, item_embedding,
           W_user, b_user, W_item, b_item):
    B = user_ids.shape[0]
    uids = user_ids.astype(jnp.int32)
    iids = item_ids.astype(jnp.int32)
    kpw0 = SPLIT // (NW * CHUNK)
    kpw1 = (B - SPLIT) // (NW * CHUNK)

    X0 = _sc_gather_piece(uids[:SPLIT].reshape(NW, kpw0, CHUNK),
                          iids[:SPLIT].reshape(NW, kpw0, CHUNK),
                          user_embedding, item_embedding)
    X1 = _sc_gather_piece(uids[SPLIT:].reshape(NW, kpw1, CHUNK),
                          iids[SPLIT:].reshape(NW, kpw1, CHUNK),
                          user_embedding, item_embedding)
    out0 = _tc_project_piece(X0, W_user, b_user, W_item, b_item, 0, B)
    ou, oi = _tc_project_piece(X1, W_user, b_user, W_item, b_item,
                               SPLIT // 2048, B, prev=out0)
    return (ou, oi)
